# Initial kernel scaffold; baseline (speedup 1.0000x reference)
import jax
import jax.numpy as jnp
from jax.experimental import pallas as pl


def kernel(event_ids, word_ids, event_table, word_table):
    raise NotImplementedError("write your pallas kernel here")


# speedup vs baseline: 2.3361x; 2.3361x over previous
import functools

import jax
import jax.numpy as jnp
from jax import lax
from jax.experimental import pallas as pl
from jax.experimental.pallas import tpu as pltpu
from jax.experimental.pallas import tpu_sc as plsc

EVENT_DIM = 32
WORD_DIM = 64

NC = 2
NS = 16
NW = NC * NS

BLK = 128
NBUF = 4


def _table_loop(tab_hbm, idx_v, rows_v, out_hbm, base, num_blocks, gsem, wsem):
    for b in range(NBUF):
        pltpu.async_copy(tab_hbm.at[idx_v.at[b]], rows_v.at[b], gsem.at[b])

    def step(s, _):
        for b in range(NBUF):
            g = s * NBUF + b
            pltpu.make_async_copy(
                tab_hbm.at[idx_v.at[g]], rows_v.at[b], gsem.at[b]
            ).wait()
            pltpu.async_copy(
                rows_v.at[b], out_hbm.at[pl.ds(base + g * BLK, BLK)], wsem.at[b]
            ).wait()

            @pl.when(g + NBUF < num_blocks)
            def _():
                pltpu.async_copy(
                    tab_hbm.at[idx_v.at[g + NBUF]], rows_v.at[b], gsem.at[b]
                )

        return _
    lax.fori_loop(0, num_blocks // NBUF, step, None)


def _emb_kernel(total, k_per_w):
    mesh = plsc.VectorSubcoreMesh(core_axis_name="c", subcore_axis_name="s")

    @functools.partial(
        pl.kernel,
        out_type=(
            jax.ShapeDtypeStruct((total, EVENT_DIM), jnp.float32),
            jax.ShapeDtypeStruct((total, WORD_DIM), jnp.float32),
        ),
        mesh=mesh,
        compiler_params=pltpu.CompilerParams(use_tc_tiling_on_sc=False),
        scratch_types=[
            pltpu.VMEM((k_per_w, BLK), jnp.int32),
            pltpu.VMEM((k_per_w, BLK), jnp.int32),
            pltpu.VMEM((NBUF, BLK, EVENT_DIM), jnp.float32),
            pltpu.VMEM((NBUF, BLK, WORD_DIM), jnp.float32),
            pltpu.SemaphoreType.DMA((NBUF,)),
            pltpu.SemaphoreType.DMA((NBUF,)),
        ],
    )
    def k(ev_idx_hbm, wd_idx_hbm, ev_tab, wd_tab, ev_out, wd_out,
          ev_idx_v, wd_idx_v, ev_rows, wd_rows, gsem, wsem):
        wid = lax.axis_index("s") * NC + lax.axis_index("c")
        base = wid * (k_per_w * BLK)
        pltpu.sync_copy(ev_idx_hbm.at[wid], ev_idx_v)
        pltpu.sync_copy(wd_idx_hbm.at[wid], wd_idx_v)
        _table_loop(ev_tab, ev_idx_v, ev_rows, ev_out, base, k_per_w, gsem, wsem)
        _table_loop(wd_tab, wd_idx_v, wd_rows, wd_out, base, k_per_w, gsem, wsem)

    return k


def kernel(event_ids, word_ids, event_table, word_table):
    batch, hist = event_ids.shape
    total = batch * hist
    k_per_w = total // (NW * BLK)
    ev_idx = event_ids.reshape(NW, k_per_w, BLK).astype(jnp.int32)
    wd_idx = word_ids.reshape(NW, k_per_w, BLK).astype(jnp.int32)
    ev_out, wd_out = _emb_kernel(total, k_per_w)(
        ev_idx, wd_idx, event_table, word_table)
    return (
        ev_out.reshape(batch, hist, EVENT_DIM),
        wd_out.reshape(batch, hist, WORD_DIM),
    )
